# Initial kernel scaffold; baseline (speedup 1.0000x reference)
import jax
import jax.numpy as jnp
from jax.experimental import pallas as pl


def kernel(keys, query, Wk, Wq):
    raise NotImplementedError("write your pallas kernel here")


# speedup vs baseline: 1.0227x; 1.0227x over previous
import functools

import jax
import jax.numpy as jnp
from jax.experimental import pallas as pl
from jax.experimental.pallas import tpu as pltpu

B, L, E, H, NE = 1, 2048, 1024, 16, 64
HD = E // H
HE = NE // H
K = NE
BQ = 256
SCALE = 1.0 / (E ** 0.5)


def _topk_kernel(q_ref, k_ref, wk_ref, wq_ref, out_ref, acc_ref):
    wkT = wk_ref[...].T
    wqT = wq_ref[...].T

    acc_ref[...] = jnp.zeros((BQ, L), jnp.float32)

    def head_body(h, _):
        qh = q_ref[h]
        kh = k_ref[h]
        qp = jax.lax.dot(qh, wqT, precision=jax.lax.Precision.HIGHEST,
                         preferred_element_type=jnp.float32)
        kp = jax.lax.dot(kh, wkT, precision=jax.lax.Precision.HIGHEST,
                         preferred_element_type=jnp.float32)
        e = jax.lax.dot_general(qp, kp, (((1,), (1,)), ((), ())),
                                precision=jax.lax.Precision.HIGHEST,
                                preferred_element_type=jnp.float32)
        e = e * SCALE
        m = jnp.max(e, axis=1, keepdims=True)
        p = jnp.exp(e - m)
        s = jnp.sum(p, axis=1, keepdims=True)
        acc_ref[...] = acc_ref[...] + p / s
        return 0

    jax.lax.fori_loop(0, H, head_body, 0, unroll=False)

    col = jax.lax.broadcasted_iota(jnp.int32, (BQ, L), 1)

    vals = acc_ref[...]
    for t in range(K):
        m = jnp.max(vals, axis=1, keepdims=True)
        cand = jnp.where(vals == m, col, L)
        idx = jnp.min(cand, axis=1, keepdims=True)
        out_ref[:, t:t + 1] = idx
        vals = jnp.where(col == idx, -1.0, vals)

    idx_mat = out_ref[...]
    lt = (idx_mat[:, :, None] < idx_mat[:, None, :]).astype(jnp.int32)
    rank = jnp.sum(lt, axis=1)
    p_iota = jax.lax.broadcasted_iota(jnp.int32, (BQ, K, K), 2)
    onehot = rank[:, :, None] == p_iota
    out_ref[...] = jnp.sum(jnp.where(onehot, idx_mat[:, :, None], 0), axis=1)


@functools.partial(jax.jit, static_argnames=("interpret",))
def kernel(keys, query, Wk, Wq, interpret=False):
    b = query.shape[0]
    outs = []
    for i in range(b):
        qh = query[i].reshape(L, H, HD).transpose(1, 0, 2)
        kh = keys[i].reshape(L, H, HD).transpose(1, 0, 2)
        edges = pl.pallas_call(
            _topk_kernel,
            grid=(L // BQ,),
            in_specs=[
                pl.BlockSpec((H, BQ, HD), lambda i: (0, i, 0)),
                pl.BlockSpec((H, L, HD), lambda i: (0, 0, 0)),
                pl.BlockSpec((HD, HD), lambda i: (0, 0)),
                pl.BlockSpec((HD, HD), lambda i: (0, 0)),
            ],
            out_specs=pl.BlockSpec((BQ, K), lambda i: (i, 0)),
            out_shape=jax.ShapeDtypeStruct((L, K), jnp.int32),
            scratch_shapes=[pltpu.VMEM((BQ, L), jnp.float32)],
            interpret=interpret,
        )(qh, kh, Wk, Wq)
        outs.append(edges)
    edges_all = jnp.stack(outs, axis=0)
    node = jnp.broadcast_to(
        jnp.arange(L, dtype=jnp.int32)[None, :, None], (b, L, K))
    edge_index = jnp.stack(
        [node.reshape(b, -1), edges_all.reshape(b, -1)], axis=1)
    return edge_index.astype(jnp.int64)


# speedup vs baseline: 1.3266x; 1.2972x over previous
import functools

import jax
import jax.numpy as jnp
from jax.experimental import pallas as pl
from jax.experimental.pallas import tpu as pltpu

B, L, E, H, NE = 1, 2048, 1024, 16, 64
HD = E // H
HE = NE // H
K = NE
BQ = 256
SCALE = 1.0 / (E ** 0.5)


def _topk_kernel(q_ref, k_ref, wk_ref, wq_ref, out_ref, acc_ref, kp_ref):
    wkT = wk_ref[...].T
    wqT = wq_ref[...].T

    @pl.when(pl.program_id(0) == 0)
    def _project_keys():
        def kp_body(h, _):
            kp_ref[h] = jax.lax.dot(k_ref[h], wkT,
                                    precision=jax.lax.Precision.HIGHEST,
                                    preferred_element_type=jnp.float32)
            return 0
        jax.lax.fori_loop(0, H, kp_body, 0, unroll=False)

    acc_ref[...] = jnp.zeros((BQ, L), jnp.float32)

    def head_body(h, _):
        qh = q_ref[h]
        qp = jax.lax.dot(qh, wqT, precision=jax.lax.Precision.HIGHEST,
                         preferred_element_type=jnp.float32)
        kp = kp_ref[h]
        e = jax.lax.dot_general(qp, kp, (((1,), (1,)), ((), ())),
                                precision=jax.lax.Precision.HIGHEST,
                                preferred_element_type=jnp.float32)
        e = e * SCALE
        m = jnp.max(e, axis=1, keepdims=True)
        p = jnp.exp(e - m)
        s = jnp.sum(p, axis=1, keepdims=True)
        acc_ref[...] = acc_ref[...] + p / s
        return 0

    jax.lax.fori_loop(0, H, head_body, 0, unroll=False)

    col = jax.lax.broadcasted_iota(jnp.int32, (BQ, L), 1)

    vals = acc_ref[...]
    for t in range(K):
        m = jnp.max(vals, axis=1, keepdims=True)
        cand = jnp.where(vals == m, col, L)
        idx = jnp.min(cand, axis=1, keepdims=True)
        out_ref[:, t:t + 1] = idx
        vals = jnp.where(col == idx, -1.0, vals)

    idx_mat = out_ref[...]
    lt = (idx_mat[:, :, None] < idx_mat[:, None, :]).astype(jnp.int32)
    rank = jnp.sum(lt, axis=1)
    p_iota = jax.lax.broadcasted_iota(jnp.int32, (BQ, K, K), 2)
    onehot = rank[:, :, None] == p_iota
    out_ref[...] = jnp.sum(jnp.where(onehot, idx_mat[:, :, None], 0), axis=1)


@functools.partial(jax.jit, static_argnames=("interpret",))
def kernel(keys, query, Wk, Wq, interpret=False):
    b = query.shape[0]
    outs = []
    for i in range(b):
        qh = query[i].reshape(L, H, HD).transpose(1, 0, 2)
        kh = keys[i].reshape(L, H, HD).transpose(1, 0, 2)
        edges = pl.pallas_call(
            _topk_kernel,
            grid=(L // BQ,),
            in_specs=[
                pl.BlockSpec((H, BQ, HD), lambda i: (0, i, 0)),
                pl.BlockSpec((H, L, HD), lambda i: (0, 0, 0)),
                pl.BlockSpec((HD, HD), lambda i: (0, 0)),
                pl.BlockSpec((HD, HD), lambda i: (0, 0)),
            ],
            out_specs=pl.BlockSpec((BQ, K), lambda i: (i, 0)),
            out_shape=jax.ShapeDtypeStruct((L, K), jnp.int32),
            scratch_shapes=[pltpu.VMEM((BQ, L), jnp.float32),
                            pltpu.VMEM((H, L, HD), jnp.float32)],
            interpret=interpret,
        )(qh, kh, Wk, Wq)
        outs.append(edges)
    edges_all = jnp.stack(outs, axis=0)
    node = jnp.broadcast_to(
        jnp.arange(L, dtype=jnp.int32)[None, :, None], (b, L, K))
    edge_index = jnp.stack(
        [node.reshape(b, -1), edges_all.reshape(b, -1)], axis=1)
    return edge_index.astype(jnp.int64)


# speedup vs baseline: 1.6504x; 1.2441x over previous
import functools

import jax
import jax.numpy as jnp
from jax.experimental import pallas as pl
from jax.experimental.pallas import tpu as pltpu

B, L, E, H, NE = 1, 2048, 1024, 16, 64
HD = E // H
HE = NE // H
K = NE
BQ = 256
SCALE = 1.0 / (E ** 0.5)


def _topk_kernel(q_ref, k_ref, wk_ref, wq_ref, out_ref, acc_ref, kp_ref):
    wkT = wk_ref[...].T
    wqT = wq_ref[...].T

    @pl.when(pl.program_id(0) == 0)
    def _project_keys():
        def kp_body(h, _):
            kp_ref[h] = jax.lax.dot(k_ref[h], wkT,
                                    precision=jax.lax.Precision.DEFAULT,
                                    preferred_element_type=jnp.float32)
            return 0
        jax.lax.fori_loop(0, H, kp_body, 0, unroll=False)

    acc_ref[...] = jnp.zeros((BQ, L), jnp.float32)

    def head_body(h, _):
        qh = q_ref[h]
        qp = jax.lax.dot(qh, wqT, precision=jax.lax.Precision.DEFAULT,
                         preferred_element_type=jnp.float32)
        kp = kp_ref[h]
        e = jax.lax.dot_general(qp, kp, (((1,), (1,)), ((), ())),
                                precision=jax.lax.Precision.DEFAULT,
                                preferred_element_type=jnp.float32)
        e = e * SCALE
        m = jnp.max(e, axis=1, keepdims=True)
        p = jnp.exp(e - m)
        s = jnp.sum(p, axis=1, keepdims=True)
        acc_ref[...] = acc_ref[...] + p / s
        return 0

    jax.lax.fori_loop(0, H, head_body, 0, unroll=False)

    col = jax.lax.broadcasted_iota(jnp.int32, (BQ, L), 1)

    vals = acc_ref[...]
    for t in range(K):
        m = jnp.max(vals, axis=1, keepdims=True)
        cand = jnp.where(vals == m, col, L)
        idx = jnp.min(cand, axis=1, keepdims=True)
        out_ref[:, t:t + 1] = idx
        vals = jnp.where(col == idx, -1.0, vals)

    idx_mat = out_ref[...]
    lt = (idx_mat[:, :, None] < idx_mat[:, None, :]).astype(jnp.int32)
    rank = jnp.sum(lt, axis=1)
    p_iota = jax.lax.broadcasted_iota(jnp.int32, (BQ, K, K), 2)
    onehot = rank[:, :, None] == p_iota
    out_ref[...] = jnp.sum(jnp.where(onehot, idx_mat[:, :, None], 0), axis=1)


@functools.partial(jax.jit, static_argnames=("interpret",))
def kernel(keys, query, Wk, Wq, interpret=False):
    b = query.shape[0]
    outs = []
    for i in range(b):
        qh = query[i].reshape(L, H, HD).transpose(1, 0, 2)
        kh = keys[i].reshape(L, H, HD).transpose(1, 0, 2)
        edges = pl.pallas_call(
            _topk_kernel,
            grid=(L // BQ,),
            in_specs=[
                pl.BlockSpec((H, BQ, HD), lambda i: (0, i, 0)),
                pl.BlockSpec((H, L, HD), lambda i: (0, 0, 0)),
                pl.BlockSpec((HD, HD), lambda i: (0, 0)),
                pl.BlockSpec((HD, HD), lambda i: (0, 0)),
            ],
            out_specs=pl.BlockSpec((BQ, K), lambda i: (i, 0)),
            out_shape=jax.ShapeDtypeStruct((L, K), jnp.int32),
            scratch_shapes=[pltpu.VMEM((BQ, L), jnp.float32),
                            pltpu.VMEM((H, L, HD), jnp.float32)],
            interpret=interpret,
        )(qh, kh, Wk, Wq)
        outs.append(edges)
    edges_all = jnp.stack(outs, axis=0)
    node = jnp.broadcast_to(
        jnp.arange(L, dtype=jnp.int32)[None, :, None], (b, L, K))
    edge_index = jnp.stack(
        [node.reshape(b, -1), edges_all.reshape(b, -1)], axis=1)
    return edge_index.astype(jnp.int64)


# speedup vs baseline: 1.9351x; 1.1725x over previous
import functools

import jax
import jax.numpy as jnp
from jax import lax
from jax.experimental import pallas as pl
from jax.experimental.pallas import tpu as pltpu

B, L, E, H, NE = 1, 2048, 1024, 16, 64
HD = E // H
K = NE
BQ = 256
SCALE = 1.0 / (E ** 0.5)


def _topk_kernel(q_ref, k_ref, wk_ref, wq_ref, out_ref, kp_ref, tri_ref):
    wkT = wk_ref[...].T
    wqT = wq_ref[...].T

    @pl.when(pl.program_id(0) == 0)
    def _init_persistent():
        def kp_body(h, _):
            kp_ref[h] = lax.dot(k_ref[h], wkT,
                                preferred_element_type=jnp.float32)
            return 0
        lax.fori_loop(0, H, kp_body, 0, unroll=False)
        r_iota = lax.broadcasted_iota(jnp.int32, (L, L), 0)
        c_iota = lax.broadcasted_iota(jnp.int32, (L, L), 1)
        tri_ref[...] = (r_iota < c_iota).astype(jnp.bfloat16)

    def head_body(h, acc):
        qp = lax.dot(q_ref[h], wqT, preferred_element_type=jnp.float32)
        e = lax.dot_general(qp, kp_ref[h], (((1,), (1,)), ((), ())),
                            preferred_element_type=jnp.float32)
        e = e * SCALE
        m = jnp.max(e, axis=1, keepdims=True)
        p = jnp.exp(e - m)
        s = jnp.sum(p, axis=1, keepdims=True)
        return acc + p / s

    acc = lax.fori_loop(0, H, head_body, jnp.zeros((BQ, L), jnp.float32),
                        unroll=False)

    bits = lax.bitcast_convert_type(acc, jnp.int32)

    lo = jnp.zeros((BQ, 1), jnp.int32)
    hi = jnp.full((BQ, 1), jnp.int32(0x7F000000))

    def bis_body(_, carry):
        lo, hi = carry
        mid = (lo + hi) >> 1
        cnt = jnp.sum((bits >= mid).astype(jnp.int32), axis=1, keepdims=True)
        ok = cnt >= K
        return jnp.where(ok, mid, lo), jnp.where(ok, hi, mid)

    lo, hi = lax.fori_loop(0, 31, bis_body, (lo, hi), unroll=False)
    T = lo

    gt = bits > T
    eq = bits == T
    need = K - jnp.sum(gt.astype(jnp.int32), axis=1, keepdims=True)
    ord_excl = lax.dot_general(eq.astype(jnp.bfloat16), tri_ref[...],
                               (((1,), (0,)), ((), ())),
                               preferred_element_type=jnp.float32)
    take = gt | (eq & (ord_excl < need.astype(jnp.float32)))

    col = lax.broadcasted_iota(jnp.int32, (BQ, L), 1)
    keys = jnp.where(take, col, jnp.int32(2 * L))
    for t in range(K):
        m = jnp.min(keys, axis=1, keepdims=True)
        out_ref[:, t:t + 1] = m
        keys = jnp.where(keys == m, jnp.int32(2 * L), keys)


@functools.partial(jax.jit, static_argnames=("interpret",))
def kernel(keys, query, Wk, Wq, interpret=False):
    b = query.shape[0]
    outs = []
    for i in range(b):
        qh = query[i].reshape(L, H, HD).transpose(1, 0, 2)
        kh = keys[i].reshape(L, H, HD).transpose(1, 0, 2)
        edges = pl.pallas_call(
            _topk_kernel,
            grid=(L // BQ,),
            in_specs=[
                pl.BlockSpec((H, BQ, HD), lambda i: (0, i, 0)),
                pl.BlockSpec((H, L, HD), lambda i: (0, 0, 0)),
                pl.BlockSpec((HD, HD), lambda i: (0, 0)),
                pl.BlockSpec((HD, HD), lambda i: (0, 0)),
            ],
            out_specs=pl.BlockSpec((BQ, K), lambda i: (i, 0)),
            out_shape=jax.ShapeDtypeStruct((L, K), jnp.int32),
            scratch_shapes=[pltpu.VMEM((H, L, HD), jnp.float32),
                            pltpu.VMEM((L, L), jnp.bfloat16)],
            interpret=interpret,
        )(qh, kh, Wk, Wq)
        outs.append(edges)
    edges_all = jnp.stack(outs, axis=0)
    node = jnp.broadcast_to(
        jnp.arange(L, dtype=jnp.int32)[None, :, None], (b, L, K))
    edge_index = jnp.stack(
        [node.reshape(b, -1), edges_all.reshape(b, -1)], axis=1)
    return edge_index.astype(jnp.int64)
